# baseline (speedup 1.0000x reference)
import functools

import jax
import jax.numpy as jnp
from jax import lax
from jax.experimental import pallas as pl
from jax.experimental.pallas import tpu as pltpu
from jax.experimental.pallas import tpu_sc as plsc

NC = 2
NS = 16
NW = NC * NS

BATCH = 16384
D_GMF = 64
D_MLP = 256
N_ROWS = 100000

B_PER_W = BATCH // NW
CHUNK = 64
N_CHUNKS = B_PER_W // CHUNK

TC_TILE = 1024
TR_TILE = 4096


def _sc_mesh():
    return plsc.VectorSubcoreMesh(
        core_axis_name="c", subcore_axis_name="s", num_cores=NC,
        num_subcores=NS)


def _sc_gather_all(users, items, Wum, Wim, Wgmf):
    out_type = (
        jax.ShapeDtypeStruct((BATCH, D_MLP), jnp.float32),
        jax.ShapeDtypeStruct((BATCH, D_MLP), jnp.float32),
        jax.ShapeDtypeStruct((BATCH, 2 * D_GMF), jnp.float32),
        jax.ShapeDtypeStruct((BATCH, 2 * D_GMF), jnp.float32),
    )
    dims = (D_MLP, D_MLP, 2 * D_GMF, 2 * D_GMF)
    scratch_types = (
        [pltpu.VMEM((B_PER_W,), jnp.int32)] * 2
        + [pltpu.VMEM((CHUNK, d), jnp.float32) for d in dims for _ in (0, 1)]
        + [pltpu.SemaphoreType.DMA] * 4
        + [pltpu.SemaphoreType.DMA] * 8
    )

    @functools.partial(pl.kernel, out_type=out_type, mesh=_sc_mesh(),
                       scratch_types=scratch_types)
    def k(u_hbm, i_hbm, wum_hbm, wim_hbm, wg_hbm,
          eum_hbm, eim_hbm, gu_hbm, gi_hbm,
          idx_u, idx_i,
          um0, um1, im0, im1, gu0, gu1, gi0, gi1,
          gs0, gs1, gs2, gs3,
          ws00, ws01, ws10, ws11, ws20, ws21, ws30, ws31):
        wid = lax.axis_index("s") * NC + lax.axis_index("c")
        base = wid * B_PER_W
        pltpu.sync_copy(u_hbm.at[pl.ds(base, B_PER_W)], idx_u)
        pltpu.sync_copy(i_hbm.at[pl.ds(base, B_PER_W)], idx_i)

        tables = (wum_hbm, wim_hbm, wg_hbm, wg_hbm)
        outs = (eum_hbm, eim_hbm, gu_hbm, gi_hbm)
        bufs = ((um0, um1), (im0, im1), (gu0, gu1), (gi0, gi1))
        gsems = (gs0, gs1, gs2, gs3)
        wsems = ((ws00, ws01), (ws10, ws11), (ws20, ws21), (ws30, ws31))

        def start_gather(c):
            s = c % 2
            iu = idx_u.at[pl.ds(c * CHUNK, CHUNK)]
            ii = idx_i.at[pl.ds(c * CHUNK, CHUNK)]
            idxs = (iu, ii, iu, ii)
            return [pltpu.async_copy(tables[t].at[idxs[t]], bufs[t][s],
                                     gsems[t]) for t in range(4)]

        def start_write(c):
            s = c % 2
            row = base + c * CHUNK
            return [pltpu.async_copy(bufs[t][s], outs[t].at[pl.ds(row, CHUNK)],
                                     wsems[t][s]) for t in range(4)]

        g = start_gather(0)
        w_prev = None
        for c in range(N_CHUNKS):
            for cp in g:
                cp.wait()
            w = start_write(c)
            if c + 1 < N_CHUNKS:
                if w_prev is not None:
                    for cp in w_prev:
                        cp.wait()
                w_prev = w
                g = start_gather(c + 1)
            else:
                for cp in w:
                    cp.wait()
                if w_prev is not None:
                    for cp in w_prev:
                        cp.wait()

    return k(users, items, Wum, Wim, Wgmf)


def _trc_body(at_ref, bt_ref, o_ref):
    o_ref[:, :D_GMF] = jnp.transpose(at_ref[...], (1, 0))
    o_ref[:, D_GMF:] = jnp.transpose(bt_ref[...], (1, 0))


def _tc_transpose_concat(WugT, WigT):
    n = WugT.shape[1]
    grid = (pl.cdiv(n, TR_TILE),)
    return pl.pallas_call(
        _trc_body,
        grid=grid,
        in_specs=[
            pl.BlockSpec((D_GMF, TR_TILE), lambda i: (0, i)),
            pl.BlockSpec((D_GMF, TR_TILE), lambda i: (0, i)),
        ],
        out_specs=pl.BlockSpec((TR_TILE, 2 * D_GMF), lambda i: (i, 0)),
        out_shape=jax.ShapeDtypeStruct((n, 2 * D_GMF), jnp.float32),
    )(WugT, WigT)


def _tc_mlp_body(eum_ref, eim_ref, gu_ref, gi_ref,
                 w1a_ref, w1b_ref, b1_ref, w2_ref, b2_ref, w3_ref, b3_ref,
                 wpg_ref, wpx_ref, bp_ref, out_ref):
    f32 = jnp.float32
    h1 = (jnp.dot(eum_ref[...], w1a_ref[...], preferred_element_type=f32)
          + jnp.dot(eim_ref[...], w1b_ref[...], preferred_element_type=f32)
          + b1_ref[...])
    h1 = jnp.maximum(h1, 0.0)
    h2 = jnp.maximum(
        jnp.dot(h1, w2_ref[...], preferred_element_type=f32) + b2_ref[...], 0.0)
    h3 = jnp.maximum(
        jnp.dot(h2, w3_ref[...], preferred_element_type=f32) + b3_ref[...], 0.0)
    g = gu_ref[:, :D_GMF] * gi_ref[:, D_GMF:]
    p = (jnp.dot(g, wpg_ref[...], preferred_element_type=f32)
         + jnp.dot(h3, wpx_ref[...], preferred_element_type=f32)
         + bp_ref[...])
    out_ref[...] = jax.nn.sigmoid(p)


def _tc_mlp(eum, eim, gu, gi, w1a, w1b, b1, w2, b2, w3, b3, wpg, wpx, bp):
    n = eum.shape[0]
    grid = (n // TC_TILE,)
    full = lambda i: (0, 0)
    return pl.pallas_call(
        _tc_mlp_body,
        grid=grid,
        in_specs=[
            pl.BlockSpec((TC_TILE, D_MLP), lambda i: (i, 0)),
            pl.BlockSpec((TC_TILE, D_MLP), lambda i: (i, 0)),
            pl.BlockSpec((TC_TILE, 2 * D_GMF), lambda i: (i, 0)),
            pl.BlockSpec((TC_TILE, 2 * D_GMF), lambda i: (i, 0)),
            pl.BlockSpec((D_MLP, D_MLP), full),
            pl.BlockSpec((D_MLP, D_MLP), full),
            pl.BlockSpec((1, D_MLP), full),
            pl.BlockSpec((D_MLP, D_MLP // 2), full),
            pl.BlockSpec((1, D_MLP // 2), full),
            pl.BlockSpec((D_MLP // 2, D_GMF), full),
            pl.BlockSpec((1, D_GMF), full),
            pl.BlockSpec((D_GMF, 1), full),
            pl.BlockSpec((D_GMF, 1), full),
            pl.BlockSpec((1, 1), full),
        ],
        out_specs=pl.BlockSpec((TC_TILE, 1), lambda i: (i, 0)),
        out_shape=jax.ShapeDtypeStruct((n, 1), jnp.float32),
    )(eum, eim, gu, gi, w1a, w1b, b1, w2, b2, w3, b3, wpg, wpx, bp)


def kernel(users, items, Wug, Wig, Wum, Wim, W1, b1, W2, b2, W3, b3, Wp, bp):
    wgmf = _tc_transpose_concat(Wug.T, Wig.T)
    eum, eim, gu, gi = _sc_gather_all(users, items, Wum, Wim, wgmf)
    w1a = W1[:D_MLP]
    w1b = W1[D_MLP:]
    wpg = Wp[:D_GMF]
    wpx = Wp[D_GMF:]
    return _tc_mlp(eum, eim, gu, gi,
                   w1a, w1b, b1.reshape(1, -1),
                   W2, b2.reshape(1, -1), W3, b3.reshape(1, -1),
                   wpg, wpx, bp.reshape(1, 1))
